# baseline (speedup 1.0000x reference)
import functools

import jax
import jax.numpy as jnp
from jax import lax
from jax.experimental import pallas as pl
from jax.experimental.pallas import tpu as pltpu
from jax.experimental.pallas import tpu_sc as plsc

NC, NS, L = 2, 16, 16
NW = NC * NS
C = 8


def _mean_agg(B, S, D):
  RPW = -(-(-(-B // NW)) // C) * C
  NCH = RPW // C
  NCH += NCH % 2
  assert D % L == 0 and C <= RPW <= B and B % C == 0 and NCH * C == RPW
  half = (C * S) // 2

  mesh = plsc.VectorSubcoreMesh(core_axis_name="c", subcore_axis_name="s")

  @functools.partial(
      pl.kernel,
      out_type=jax.ShapeDtypeStruct((B, D), jnp.float32),
      mesh=mesh,
      scratch_types=[
          pltpu.VMEM((RPW * S,), jnp.int32),
          pltpu.VMEM((C * S, D), jnp.float32),
          pltpu.VMEM((C * S, D), jnp.float32),
          pltpu.VMEM((C, D), jnp.float32),
          pltpu.VMEM((L,), jnp.float32),
          pltpu.SemaphoreType.DMA,
          pltpu.SemaphoreType.DMA,
      ],
  )
  def k(x_hbm, neigh_hbm, scale_hbm, out_hbm, idx_all, gb0, gb1, outb, scl,
        sem0, sem1):
    wid = lax.axis_index("s") * NC + lax.axis_index("c")
    base_w = jnp.minimum(wid * RPW, B - RPW)
    pltpu.sync_copy(scale_hbm, scl)
    pltpu.sync_copy(neigh_hbm.at[pl.ds(base_w * S, RPW * S)], idx_all)

    def chunk_off(g):
      return g * C

    def issue(g, gath_v, sem):
      off = chunk_off(g) * S
      pltpu.async_copy(x_hbm.at[idx_all.at[pl.ds(off, half)]],
                       gath_v.at[pl.ds(0, half)], sem)
      pltpu.async_copy(x_hbm.at[idx_all.at[pl.ds(off + half, half)]],
                       gath_v.at[pl.ds(half, half)], sem)

    def wait(gath_v, sem):
      pltpu.make_async_copy(x_hbm.at[pl.ds(0, C * S)], gath_v, sem).wait()

    def accumulate(gath_v):
      scale = scl[...]

      def row(r, carry):
        rb = r * S
        for j in range(D // L):
          acc = gath_v[rb, pl.ds(j * L, L)]
          for s in range(1, S):
            acc = acc + gath_v[rb + s, pl.ds(j * L, L)]
          outb[r, pl.ds(j * L, L)] = acc * scale
        return carry

      lax.fori_loop(0, C, row, 0)

    def do_chunk(g, has_next, gath_v, sem):
      accumulate(gath_v)


      pltpu.sync_copy(outb, out_hbm.at[pl.ds(base_w + chunk_off(g), C)])


    def loop(g2, carry):
      g = g2 * 2
      do_chunk(g, g + 2 < NCH, gb0, sem0)
      do_chunk(g + 1, g + 3 < NCH, gb1, sem1)
      return carry

    lax.fori_loop(0, NCH // 2, loop, 0)

  return k


def kernel(x, nodes, neigh_nodes, num_sample):
  del nodes
  B, S = neigh_nodes.shape
  _, D = x.shape
  scale = jnp.full((L,), 1.0, jnp.float32) / jnp.asarray(num_sample, jnp.float32)
  return _mean_agg(B, S, D)(x, neigh_nodes.reshape(-1), scale)


# speedup vs baseline: 1.1812x; 1.1812x over previous
import functools

import jax
import jax.numpy as jnp
from jax import lax
from jax.experimental import pallas as pl
from jax.experimental.pallas import tpu as pltpu
from jax.experimental.pallas import tpu_sc as plsc

NC, NS, L = 2, 16, 16
NW = NC * NS
C = 8


def _mean_agg(B, S, D):
  RPW = -(-(-(-B // NW)) // C) * C
  NCH = RPW // C
  NCH += NCH % 2
  assert D % L == 0 and C <= RPW <= B and B % C == 0 and NCH * C == RPW
  half = (C * S) // 2

  mesh = plsc.VectorSubcoreMesh(core_axis_name="c", subcore_axis_name="s")

  @functools.partial(
      pl.kernel,
      out_type=jax.ShapeDtypeStruct((B, D), jnp.float32),
      mesh=mesh,
      scratch_types=[
          pltpu.VMEM((RPW * S,), jnp.int32),
          pltpu.VMEM((C * S, D), jnp.float32),
          pltpu.VMEM((C * S, D), jnp.float32),
          pltpu.VMEM((C, D), jnp.float32),
          pltpu.VMEM((C, D), jnp.float32),
          pltpu.VMEM((L,), jnp.float32),
          pltpu.SemaphoreType.DMA,
          pltpu.SemaphoreType.DMA,
          pltpu.SemaphoreType.DMA,
          pltpu.SemaphoreType.DMA,
      ],
  )
  def k(x_hbm, neigh_hbm, scale_hbm, out_hbm, idx_all, gb0, gb1, ob0, ob1,
        scl, sem0, sem1, osem0, osem1):
    wid = lax.axis_index("s") * NC + lax.axis_index("c")
    base_w = jnp.minimum(wid * RPW, B - RPW)
    pltpu.sync_copy(scale_hbm, scl)
    pltpu.sync_copy(neigh_hbm.at[pl.ds(base_w * S, RPW * S)], idx_all)

    def issue(g, gath_v, sem):
      off = g * (C * S)
      pltpu.async_copy(x_hbm.at[idx_all.at[pl.ds(off, half)]],
                       gath_v.at[pl.ds(0, half)], sem)
      pltpu.async_copy(x_hbm.at[idx_all.at[pl.ds(off + half, half)]],
                       gath_v.at[pl.ds(half, half)], sem)

    def wait_gather(gath_v, sem):
      pltpu.make_async_copy(x_hbm.at[pl.ds(0, C * S)], gath_v, sem).wait()

    def wait_out(outb, osem):
      pltpu.make_async_copy(outb, out_hbm.at[pl.ds(0, C)], osem).wait()

    def accumulate(gath_v, outb):
      scale = scl[...]

      def row(r, carry):
        rb = r * S
        accs = [gath_v[rb, pl.ds(j * L, L)] for j in range(D // L)]
        for s in range(1, S):
          for j in range(D // L):
            accs[j] = accs[j] + gath_v[rb + s, pl.ds(j * L, L)]
        for j in range(D // L):
          outb[r, pl.ds(j * L, L)] = accs[j] * scale
        return carry

      lax.fori_loop(0, C, row, 0)

    def do_chunk(g, has_next, gath_v, sem, outb, osem):
      wait_gather(gath_v, sem)

      @pl.when(g >= 2)
      def _():
        wait_out(outb, osem)

      accumulate(gath_v, outb)

      @pl.when(has_next)
      def _():
        issue(g + 2, gath_v, sem)

      pltpu.async_copy(outb, out_hbm.at[pl.ds(base_w + g * C, C)], osem)

    issue(0, gb0, sem0)
    issue(1, gb1, sem1)

    def loop(g2, carry):
      g = g2 * 2
      do_chunk(g, g + 2 < NCH, gb0, sem0, ob0, osem0)
      do_chunk(g + 1, g + 3 < NCH, gb1, sem1, ob1, osem1)
      return carry

    lax.fori_loop(0, NCH // 2, loop, 0)
    wait_out(ob0, osem0)
    wait_out(ob1, osem1)

  return k


def kernel(x, nodes, neigh_nodes, num_sample):
  del nodes
  B, S = neigh_nodes.shape
  _, D = x.shape
  scale = jnp.full((L,), 1.0, jnp.float32) / jnp.asarray(num_sample, jnp.float32)
  return _mean_agg(B, S, D)(x, neigh_nodes.reshape(-1), scale)


# speedup vs baseline: 1.4849x; 1.2571x over previous
import functools

import jax
import jax.numpy as jnp
from jax import lax
from jax.experimental import pallas as pl
from jax.experimental.pallas import tpu as pltpu
from jax.experimental.pallas import tpu_sc as plsc

NC, NS, L = 2, 16, 16
NW = NC * NS
C = 4
NBUF = 4


def _mean_agg(B, S, D):
  RPW = -(-(-(-B // NW)) // 8) * 8
  NCH = RPW // C
  CS = C * S
  assert D % L == 0 and C <= RPW <= B and B % 8 == 0
  assert NCH % NBUF == 0 and NBUF % 2 == 0 and CS <= 128 and CS % 8 == 0

  mesh = plsc.VectorSubcoreMesh(core_axis_name="c", subcore_axis_name="s")

  @functools.partial(
      pl.kernel,
      out_type=jax.ShapeDtypeStruct((B, D), jnp.float32),
      mesh=mesh,
      scratch_types=[
          pltpu.VMEM((RPW * S,), jnp.int32),
          *[pltpu.VMEM((CS, D), jnp.float32) for _ in range(NBUF)],
          pltpu.VMEM((2 * C, D), jnp.float32),
          pltpu.VMEM((2 * C, D), jnp.float32),
          pltpu.VMEM((L,), jnp.float32),
          *[pltpu.SemaphoreType.DMA for _ in range(NBUF + 2)],
      ],
  )
  def k(x_hbm, neigh_hbm, scale_hbm, out_hbm, idx_all, *rest):
    gbs = rest[:NBUF]
    obA, obB, scl = rest[NBUF:NBUF + 3]
    sems = rest[NBUF + 3:NBUF + 3 + NBUF]
    osemA, osemB = rest[NBUF + 3 + NBUF:]
    wid = lax.axis_index("s") * NC + lax.axis_index("c")
    base_w = jnp.minimum(wid * RPW, B - RPW)
    pltpu.sync_copy(scale_hbm, scl)
    pltpu.sync_copy(neigh_hbm.at[pl.ds(base_w * S, RPW * S)], idx_all)

    def gather(g, gath_v, sem):
      return pltpu.make_async_copy(x_hbm.at[idx_all.at[pl.ds(g * CS, CS)]],
                                   gath_v, sem)

    def wait_out(outb, osem):
      pltpu.make_async_copy(outb, out_hbm.at[pl.ds(0, 2 * C)], osem).wait()

    def accumulate(gath_v, outb, ro):
      scale = scl[...]

      def row(r, carry):
        rb = r * S

        def sbody(s, accs):
          return tuple(accs[j] + gath_v[rb + s, pl.ds(j * L, L)]
                       for j in range(D // L))

        accs = lax.fori_loop(
            1, S, sbody,
            tuple(gath_v[rb, pl.ds(j * L, L)] for j in range(D // L)),
            unroll=4)
        for j in range(D // L):
          outb[ro + r, pl.ds(j * L, L)] = accs[j] * scale
        return carry

      lax.fori_loop(0, C, row, 0)

    def do_chunk(g, gath_v, sem, outb, ro):
      gather(g, gath_v, sem).wait()
      accumulate(gath_v, outb, ro)

      @pl.when(g + NBUF < NCH)
      def _():
        gather(g + NBUF, gath_v, sem).start()

    for b in range(NBUF):
      gather(b, gbs[b], sems[b]).start()

    def loop(g2, carry):
      g0 = g2 * NBUF
      for p, (ob, osem) in enumerate(((obA, osemA), (obB, osemB))):
        g = g0 + 2 * p

        @pl.when(g0 >= NBUF)
        def _(ob=ob, osem=osem):
          wait_out(ob, osem)

        do_chunk(g, gbs[2 * p], sems[2 * p], ob, 0)
        do_chunk(g + 1, gbs[2 * p + 1], sems[2 * p + 1], ob, C)
        pltpu.async_copy(ob, out_hbm.at[pl.ds(base_w + g * C, 2 * C)], osem)
      return carry

    lax.fori_loop(0, NCH // NBUF, loop, 0)
    wait_out(obA, osemA)
    wait_out(obB, osemB)

  return k


def kernel(x, nodes, neigh_nodes, num_sample):
  del nodes
  B, S = neigh_nodes.shape
  _, D = x.shape
  scale = jnp.full((L,), 1.0, jnp.float32) / jnp.asarray(num_sample, jnp.float32)
  return _mean_agg(B, S, D)(x, neigh_nodes.reshape(-1), scale)


# speedup vs baseline: 1.4880x; 1.0021x over previous
import functools

import jax
import jax.numpy as jnp
from jax import lax
from jax.experimental import pallas as pl
from jax.experimental.pallas import tpu as pltpu
from jax.experimental.pallas import tpu_sc as plsc

NC, NS, L = 2, 16, 16
NW = NC * NS
C = 4
NBUF = 4


def _mean_agg(B, S, D):
  RPW = -(-(-(-B // NW)) // 8) * 8
  NCH = RPW // C
  CS = C * S
  assert D % L == 0 and C <= RPW <= B and B % 8 == 0
  assert NCH % NBUF == 0 and NBUF % 2 == 0 and CS <= 128 and CS % 8 == 0

  mesh = plsc.VectorSubcoreMesh(core_axis_name="c", subcore_axis_name="s")

  @functools.partial(
      pl.kernel,
      out_type=jax.ShapeDtypeStruct((B, D), jnp.float32),
      mesh=mesh,
      scratch_types=[
          pltpu.VMEM((RPW * S,), jnp.int32),
          *[pltpu.VMEM((CS, D), jnp.float32) for _ in range(NBUF)],
          pltpu.VMEM((2 * C, D), jnp.float32),
          pltpu.VMEM((2 * C, D), jnp.float32),
          pltpu.VMEM((L,), jnp.float32),
          *[pltpu.SemaphoreType.DMA for _ in range(NBUF + 2)],
      ],
  )
  def k(x_hbm, neigh_hbm, scale_hbm, out_hbm, idx_all, *rest):
    gbs = rest[:NBUF]
    obA, obB, scl = rest[NBUF:NBUF + 3]
    sems = rest[NBUF + 3:NBUF + 3 + NBUF]
    osemA, osemB = rest[NBUF + 3 + NBUF:]
    wid = lax.axis_index("s") * NC + lax.axis_index("c")
    base_w = jnp.minimum(wid * RPW, B - RPW)
    pltpu.sync_copy(scale_hbm, scl)
    pltpu.sync_copy(neigh_hbm.at[pl.ds(base_w * S, RPW * S)], idx_all)

    H = CS // 2

    def gather_parts(g, gath_v, sem):
      return (
          pltpu.make_async_copy(x_hbm.at[idx_all.at[pl.ds(g * CS, H)]],
                                gath_v.at[pl.ds(0, H)], sem),
          pltpu.make_async_copy(x_hbm.at[idx_all.at[pl.ds(g * CS + H, H)]],
                                gath_v.at[pl.ds(H, H)], sem),
      )

    class gather:
      def __init__(self, g, gath_v, sem):
        self.parts = gather_parts(g, gath_v, sem)

      def start(self):
        for cp in self.parts:
          cp.start()

      def wait(self):
        for cp in self.parts:
          cp.wait()

    def wait_out(outb, osem):
      pltpu.make_async_copy(outb, out_hbm.at[pl.ds(0, 2 * C)], osem).wait()

    def accumulate(gath_v, outb, ro):
      scale = scl[...]

      def row(r, carry):
        rb = r * S

        def sbody(s, accs):
          return tuple(accs[j] + gath_v[rb + s, pl.ds(j * L, L)]
                       for j in range(D // L))

        accs = lax.fori_loop(
            1, S, sbody,
            tuple(gath_v[rb, pl.ds(j * L, L)] for j in range(D // L)),
            unroll=4)
        for j in range(D // L):
          outb[ro + r, pl.ds(j * L, L)] = accs[j] * scale
        return carry

      lax.fori_loop(0, C, row, 0)

    def do_chunk(g, gath_v, sem, outb, ro):
      gather(g, gath_v, sem).wait()
      accumulate(gath_v, outb, ro)

      @pl.when(g + NBUF < NCH)
      def _():
        gather(g + NBUF, gath_v, sem).start()

    for b in range(NBUF):
      gather(b, gbs[b], sems[b]).start()

    def loop(g2, carry):
      g0 = g2 * NBUF
      for p, (ob, osem) in enumerate(((obA, osemA), (obB, osemB))):
        g = g0 + 2 * p

        @pl.when(g0 >= NBUF)
        def _(ob=ob, osem=osem):
          wait_out(ob, osem)

        do_chunk(g, gbs[2 * p], sems[2 * p], ob, 0)
        do_chunk(g + 1, gbs[2 * p + 1], sems[2 * p + 1], ob, C)
        pltpu.async_copy(ob, out_hbm.at[pl.ds(base_w + g * C, 2 * C)], osem)
      return carry

    lax.fori_loop(0, NCH // NBUF, loop, 0)
    wait_out(obA, osemA)
    wait_out(obB, osemB)

  return k


def kernel(x, nodes, neigh_nodes, num_sample):
  del nodes
  B, S = neigh_nodes.shape
  _, D = x.shape
  scale = jnp.full((L,), 1.0, jnp.float32) / jnp.asarray(num_sample, jnp.float32)
  return _mean_agg(B, S, D)(x, neigh_nodes.reshape(-1), scale)


# speedup vs baseline: 1.4881x; 1.0001x over previous
import functools

import jax
import jax.numpy as jnp
from jax import lax
from jax.experimental import pallas as pl
from jax.experimental.pallas import tpu as pltpu
from jax.experimental.pallas import tpu_sc as plsc

NC, NS, L = 2, 16, 16
NW = NC * NS
C = 4
NBUF = 4


def _mean_agg(B, S, D):
  RPW = -(-(-(-B // NW)) // 8) * 8
  NCH = RPW // C
  CS = C * S
  assert D % L == 0 and C <= RPW <= B and B % 8 == 0
  assert NCH % NBUF == 0 and NBUF % 2 == 0 and CS <= 128 and CS % 8 == 0

  mesh = plsc.VectorSubcoreMesh(core_axis_name="c", subcore_axis_name="s")

  @functools.partial(
      pl.kernel,
      out_type=jax.ShapeDtypeStruct((B, D), jnp.float32),
      mesh=mesh,
      scratch_types=[
          pltpu.VMEM((RPW * S,), jnp.int32),
          *[pltpu.VMEM((CS, D), jnp.float32) for _ in range(NBUF)],
          pltpu.VMEM((2 * C, D), jnp.float32),
          pltpu.VMEM((2 * C, D), jnp.float32),
          pltpu.VMEM((L,), jnp.float32),
          *[pltpu.SemaphoreType.DMA for _ in range(NBUF + 2)],
      ],
  )
  def k(x_hbm, neigh_hbm, scale_hbm, out_hbm, idx_all, *rest):
    gbs = rest[:NBUF]
    obA, obB, scl = rest[NBUF:NBUF + 3]
    sems = rest[NBUF + 3:NBUF + 3 + NBUF]
    osemA, osemB = rest[NBUF + 3 + NBUF:]
    wid = lax.axis_index("s") * NC + lax.axis_index("c")
    base_w = jnp.minimum(wid * RPW, B - RPW)
    pltpu.sync_copy(scale_hbm, scl)
    pltpu.sync_copy(neigh_hbm.at[pl.ds(base_w * S, RPW * S)], idx_all)

    def gather(g, gath_v, sem):
      return pltpu.make_async_copy(x_hbm.at[idx_all.at[pl.ds(g * CS, CS)]],
                                   gath_v, sem)

    def wait_out(outb, osem):
      pltpu.make_async_copy(outb, out_hbm.at[pl.ds(0, 2 * C)], osem).wait()

    def accumulate(gath_v, outb, ro):
      scale = scl[...]

      def row(r, carry):
        rb = r * S

        def sbody(s, accs):
          return tuple(accs[j] + gath_v[rb + s, pl.ds(j * L, L)]
                       for j in range(D // L))

        accs = lax.fori_loop(
            1, S, sbody,
            tuple(gath_v[rb, pl.ds(j * L, L)] for j in range(D // L)),
            unroll=4)
        for j in range(D // L):
          outb[ro + r, pl.ds(j * L, L)] = accs[j] * scale
        return carry

      lax.fori_loop(0, C, row, 0)

    def do_chunk(g, gath_v, sem, outb, ro):
      gather(g, gath_v, sem).wait()
      accumulate(gath_v, outb, ro)

      @pl.when(g + NBUF < NCH)
      def _():
        gather(g + NBUF, gath_v, sem).start()

    for b in range(NBUF):
      gather(b, gbs[b], sems[b]).start()

    def loop(g2, carry):
      g0 = g2 * NBUF
      for p, (ob, osem) in enumerate(((obA, osemA), (obB, osemB))):
        g = g0 + 2 * p

        @pl.when(g0 >= NBUF)
        def _(ob=ob, osem=osem):
          wait_out(ob, osem)

        do_chunk(g, gbs[2 * p], sems[2 * p], ob, 0)
        do_chunk(g + 1, gbs[2 * p + 1], sems[2 * p + 1], ob, C)
        pltpu.async_copy(ob, out_hbm.at[pl.ds(base_w + g * C, 2 * C)], osem)
      return carry

    lax.fori_loop(0, NCH // NBUF, loop, 0)
    wait_out(obA, osemA)
    wait_out(obB, osemB)

  return k


def kernel(x, nodes, neigh_nodes, num_sample):
  del nodes
  B, S = neigh_nodes.shape
  _, D = x.shape
  scale = jnp.full((L,), 1.0, jnp.float32) / jnp.asarray(num_sample, jnp.float32)
  return _mean_agg(B, S, D)(x, neigh_nodes.reshape(-1), scale)
